# Initial kernel scaffold; baseline (speedup 1.0000x reference)
import jax
import jax.numpy as jnp
from jax.experimental import pallas as pl


def kernel(snp_ids, is_padding, emb_table, ln_gamma, ln_beta):
    raise NotImplementedError("write your pallas kernel here")


# speedup vs baseline: 3.3689x; 3.3689x over previous
import functools

import jax
import jax.numpy as jnp
from jax.experimental import pallas as pl


def _body(ids_ref, tab_ref, gamma_ref, beta_ref, out_ref):
    tab = tab_ref[...]
    mean = jnp.mean(tab, axis=1, keepdims=True)
    var = jnp.mean((tab - mean) ** 2, axis=1, keepdims=True)
    ntab = (tab - mean) * jax.lax.rsqrt(var + 1e-12)
    ntab = ntab * gamma_ref[...] + beta_ref[...]

    ids = ids_ref[0]
    acc = jnp.where(ids == 0, ntab[0, :][None, :], ntab[1, :][None, :])
    for v in range(2, tab.shape[0]):
        acc = jnp.where(ids == v, ntab[v, :][None, :], acc)
    out_ref[...] = acc


@functools.partial(jax.jit, static_argnames=())
def kernel(snp_ids, is_padding, emb_table, ln_gamma, ln_beta):
    B, L = snp_ids.shape
    V, D = emb_table.shape
    N = B * L
    chunk = 2048
    ids3 = snp_ids.reshape(N // chunk, chunk, 1)
    out = pl.pallas_call(
        _body,
        grid=(N // chunk,),
        in_specs=[
            pl.BlockSpec((1, chunk, 1), lambda i: (i, 0, 0)),
            pl.BlockSpec((V, D), lambda i: (0, 0)),
            pl.BlockSpec((1, D), lambda i: (0, 0)),
            pl.BlockSpec((1, D), lambda i: (0, 0)),
        ],
        out_specs=pl.BlockSpec((chunk, D), lambda i: (i, 0)),
        out_shape=jax.ShapeDtypeStruct((N, D), jnp.float32),
    )(ids3, emb_table, ln_gamma.reshape(1, D), ln_beta.reshape(1, D))
    return out.reshape(B, L, D), is_padding


# speedup vs baseline: 3.6319x; 1.0781x over previous
import functools

import jax
import jax.numpy as jnp
from jax.experimental import pallas as pl

CHUNK = 2048
SUB = 64


def _body(ids_ref, tab_ref, gamma_ref, beta_ref, out_ref):
    tab = tab_ref[...]
    V, D = tab.shape
    mean = jnp.mean(tab, axis=1, keepdims=True)
    var = jnp.mean((tab - mean) ** 2, axis=1, keepdims=True)
    ntab = (tab - mean) * jax.lax.rsqrt(var + 1e-12)
    ntab = ntab * gamma_ref[...] + beta_ref[...]

    def step(i, _):
        ids = ids_ref[0, pl.ds(i * SUB, SUB), :]
        idb = jnp.broadcast_to(ids, (SUB, D))
        acc = jnp.where(idb == 0, ntab[0:1, :], ntab[1:2, :])
        for v in range(2, V):
            acc = jnp.where(idb == v, ntab[v:v + 1, :], acc)
        out_ref[pl.ds(i * SUB, SUB), :] = acc
        return 0

    jax.lax.fori_loop(0, CHUNK // SUB, step, 0, unroll=32)


@functools.partial(jax.jit, static_argnames=())
def kernel(snp_ids, is_padding, emb_table, ln_gamma, ln_beta):
    B, L = snp_ids.shape
    V, D = emb_table.shape
    N = B * L
    ids3 = snp_ids.reshape(N // CHUNK, CHUNK, 1)
    out = pl.pallas_call(
        _body,
        grid=(N // CHUNK,),
        in_specs=[
            pl.BlockSpec((1, CHUNK, 1), lambda i: (i, 0, 0)),
            pl.BlockSpec((V, D), lambda i: (0, 0)),
            pl.BlockSpec((1, D), lambda i: (0, 0)),
            pl.BlockSpec((1, D), lambda i: (0, 0)),
        ],
        out_specs=pl.BlockSpec((CHUNK, D), lambda i: (i, 0)),
        out_shape=jax.ShapeDtypeStruct((N, D), jnp.float32),
    )(ids3, emb_table, ln_gamma.reshape(1, D), ln_beta.reshape(1, D))
    return out.reshape(B, L, D), is_padding


# speedup vs baseline: 3.6449x; 1.0036x over previous
import functools

import jax
import jax.numpy as jnp
from jax.experimental import pallas as pl
from jax.experimental.pallas import tpu as pltpu

CHUNK = 2048
SUB = 64


def _body(ids_ref, tab_ref, gamma_ref, beta_ref, out_ref):
    tab = tab_ref[...]
    V, D = tab.shape
    mean = jnp.mean(tab, axis=1, keepdims=True)
    var = jnp.mean((tab - mean) ** 2, axis=1, keepdims=True)
    ntab = (tab - mean) * jax.lax.rsqrt(var + 1e-12)
    ntab = ntab * gamma_ref[...] + beta_ref[...]

    def step(i, _):
        ids = ids_ref[0, pl.ds(i * SUB, SUB), :]
        idb = jnp.broadcast_to(ids, (SUB, D))
        acc = jnp.where(idb == 0, ntab[0:1, :], ntab[1:2, :])
        for v in range(2, V):
            acc = jnp.where(idb == v, ntab[v:v + 1, :], acc)
        out_ref[pl.ds(i * SUB, SUB), :] = acc
        return 0

    jax.lax.fori_loop(0, CHUNK // SUB, step, 0, unroll=32)


@functools.partial(jax.jit, static_argnames=())
def kernel(snp_ids, is_padding, emb_table, ln_gamma, ln_beta):
    B, L = snp_ids.shape
    V, D = emb_table.shape
    N = B * L
    ids3 = snp_ids.reshape(N // CHUNK, CHUNK, 1)
    out = pl.pallas_call(
        _body,
        grid=(N // CHUNK,),
        in_specs=[
            pl.BlockSpec((1, CHUNK, 1), lambda i: (i, 0, 0)),
            pl.BlockSpec((V, D), lambda i: (0, 0)),
            pl.BlockSpec((1, D), lambda i: (0, 0)),
            pl.BlockSpec((1, D), lambda i: (0, 0)),
        ],
        out_specs=pl.BlockSpec((CHUNK, D), lambda i: (i, 0)),
        out_shape=jax.ShapeDtypeStruct((N, D), jnp.float32),
        compiler_params=pltpu.CompilerParams(
            dimension_semantics=("parallel",),
        ),
    )(ids3, emb_table, ln_gamma.reshape(1, D), ln_beta.reshape(1, D))
    return out.reshape(B, L, D), is_padding
